# baseline (speedup 1.0000x reference)
import functools

import jax
import jax.numpy as jnp
from jax import lax
from jax.experimental import pallas as pl
from jax.experimental.pallas import tpu as pltpu
from jax.experimental.pallas import tpu_sc as plsc

EMB = 64
BATCH = 4096
HIST = 200
LANES = 16
NC = 2
NS = 16
NW = NC * NS
ROWS_PER_W = BATCH // NW
IDX_PER_W = ROWS_PER_W * HIST
GROUPS = ROWS_PER_W // LANES
PAIRS = LANES // 2
C0 = 128
C1 = HIST - C0
ACC_UNROLL = 8


def _sc_body(idx_hbm, wb_hbm, table_hbm, out_hbm,
             idx_v, buf0, buf1, wb_v, probs_v, sem0, sem1):
    wid = lax.axis_index("s") * NC + lax.axis_index("c")

    base_i = pl.multiple_of(wid * IDX_PER_W, 8)
    pltpu.sync_copy(idx_hbm.at[pl.ds(base_i, IDX_PER_W)], idx_v)
    pltpu.sync_copy(wb_hbm, wb_v)

    w = [wb_v[pl.ds(k * LANES, LANES)] for k in range(EMB // LANES)]
    bias = wb_v[pl.ds(EMB, LANES)][0]

    def issue(row, buf, sem):
        start = pl.multiple_of(row * HIST, 8)
        pltpu.make_async_copy(
            table_hbm.at[idx_v.at[pl.ds(start, C0)]],
            buf.at[pl.ds(0, C0)], sem).start()
        pltpu.make_async_copy(
            table_hbm.at[idx_v.at[pl.ds(start + C0, C1)]],
            buf.at[pl.ds(C0, C1)], sem).start()

    def wait(buf, sem):
        pltpu.make_async_copy(
            table_hbm.at[idx_v.at[pl.ds(0, C0)]],
            buf.at[pl.ds(0, C0)], sem).wait()
        pltpu.make_async_copy(
            table_hbm.at[idx_v.at[pl.ds(0, C1)]],
            buf.at[pl.ds(C0, C1)], sem).wait()

    def accum(buf):
        zero = jnp.zeros((LANES,), jnp.float32)

        def body(j, accs):
            a0, a1, a2, a3 = accs
            r = j * ACC_UNROLL
            for u in range(ACC_UNROLL):
                row = r + u
                a0 = a0 + buf[row, pl.ds(0, LANES)]
                a1 = a1 + buf[row, pl.ds(LANES, LANES)]
                a2 = a2 + buf[row, pl.ds(2 * LANES, LANES)]
                a3 = a3 + buf[row, pl.ds(3 * LANES, LANES)]
            return (a0, a1, a2, a3)

        a0, a1, a2, a3 = lax.fori_loop(
            0, HIST // ACC_UNROLL, body, (zero, zero, zero, zero))
        s = a0 * w[0] + a1 * w[1] + a2 * w[2] + a3 * w[3]
        return jnp.sum(s) * (1.0 / HIST) + bias

    lane_iota = lax.iota(jnp.int32, LANES)
    last = ROWS_PER_W - 1

    issue(0, buf0, sem0)
    issue(1, buf1, sem1)
    for g in range(GROUPS):
        def pair(i, logits):
            r0 = g * LANES + 2 * i
            wait(buf0, sem0)
            z0 = accum(buf0)
            issue(jnp.minimum(r0 + 2, last), buf0, sem0)
            logits = jnp.where(lane_iota == 2 * i, z0, logits)
            wait(buf1, sem1)
            z1 = accum(buf1)
            issue(jnp.minimum(r0 + 3, last), buf1, sem1)
            logits = jnp.where(lane_iota == 2 * i + 1, z1, logits)
            return logits

        logits = lax.fori_loop(0, PAIRS, pair, jnp.zeros((LANES,), jnp.float32))
        probs_v[pl.ds(g * LANES, LANES)] = 1.0 / (1.0 + jnp.exp(-logits))

    wait(buf0, sem0)
    wait(buf1, sem1)
    base_o = pl.multiple_of(wid * ROWS_PER_W, 8)
    pltpu.sync_copy(probs_v, out_hbm.at[pl.ds(base_o, ROWS_PER_W)])


_embed_avg_sc = functools.partial(
    pl.kernel,
    mesh=plsc.VectorSubcoreMesh(core_axis_name="c", subcore_axis_name="s"),
    out_type=jax.ShapeDtypeStruct((BATCH,), jnp.float32),
    compiler_params=pltpu.CompilerParams(
        needs_layout_passes=False, use_tc_tiling_on_sc=False),
    scratch_types=[
        pltpu.VMEM((IDX_PER_W,), jnp.int32),
        pltpu.VMEM((HIST, EMB), jnp.float32),
        pltpu.VMEM((HIST, EMB), jnp.float32),
        pltpu.VMEM((EMB + LANES,), jnp.float32),
        pltpu.VMEM((ROWS_PER_W,), jnp.float32),
        pltpu.SemaphoreType.DMA,
        pltpu.SemaphoreType.DMA,
    ],
)(_sc_body)


def kernel(x, emb_table, W, b):
    wb = jnp.concatenate([
        W.reshape(-1).astype(jnp.float32),
        b.reshape(-1).astype(jnp.float32),
        jnp.zeros((LANES - 1,), jnp.float32),
    ])
    out = _embed_avg_sc(x.reshape(-1), wb, emb_table)
    return out.reshape(BATCH, 1)
